# Initial kernel scaffold; baseline (speedup 1.0000x reference)
import jax
import jax.numpy as jnp
from jax.experimental import pallas as pl


def kernel(hidden_states, gate_w, w13, w2, shared_w13, shared_w2):
    raise NotImplementedError("write your pallas kernel here")


# speedup vs baseline: 1.4840x; 1.4840x over previous
import jax
import jax.numpy as jnp
from jax import lax
from jax.experimental import pallas as pl
from jax.experimental.pallas import tpu as pltpu

T = 2048
H = 1024
F = 1024
E = 8
TOP_K = 2
N_GROUP = 4
TOPK_GROUP = 2
SF = 2048

BT = 512


def _router_coef(x_f32, gate_w):
    B = x_f32.shape[0]
    logits = jnp.dot(x_f32, gate_w.T, preferred_element_type=jnp.float32)
    s = jax.nn.softmax(logits, axis=-1)
    g = jnp.max(s.reshape(B, N_GROUP, E // N_GROUP), axis=-1)
    jidx = lax.broadcasted_iota(jnp.int32, (B, N_GROUP), 1)
    m1 = jnp.max(g, axis=-1, keepdims=True)
    i1 = jnp.min(jnp.where(g == m1, jidx, N_GROUP), axis=-1, keepdims=True)
    g2 = jnp.where(jidx == i1, -1.0, g)
    m2 = jnp.max(g2, axis=-1, keepdims=True)
    i2 = jnp.min(jnp.where(g2 == m2, jidx, N_GROUP), axis=-1, keepdims=True)
    eidx = lax.broadcasted_iota(jnp.int32, (B, E), 1)
    gid = eidx // (E // N_GROUP)
    keep = (gid == i1) | (gid == i2)
    sm = jnp.where(keep, s, 0.0)
    w1 = jnp.max(sm, axis=-1, keepdims=True)
    e1 = jnp.min(jnp.where(sm == w1, eidx, E), axis=-1, keepdims=True)
    sm2 = jnp.where(eidx == e1, -1.0, sm)
    w2v = jnp.max(sm2, axis=-1, keepdims=True)
    e2 = jnp.min(jnp.where(sm2 == w2v, eidx, E), axis=-1, keepdims=True)
    denom = w1 + w2v + 1e-20
    coef = (jnp.where(eidx == e1, w1, 0.0)
            + jnp.where(eidx == e2, w2v, 0.0)) / denom
    return coef


def _moe_body(x_ref, gw_ref, w13_ref, w2_ref, sw13_ref, sw2_ref,
              out_ref, coef_ref):
    e = pl.program_id(1)

    @pl.when(e == 0)
    def _init():
        x32 = x_ref[...]
        coef_ref[...] = _router_coef(x32, gw_ref[...])
        xb = x32.astype(jnp.bfloat16)
        h1 = jnp.dot(xb, sw13_ref[...].T, preferred_element_type=jnp.float32)
        gate, up = jnp.split(h1, 2, axis=-1)
        h2 = (jax.nn.sigmoid(gate) * gate * up).astype(jnp.bfloat16)
        out_ref[...] = jnp.dot(h2, sw2_ref[...].T,
                               preferred_element_type=jnp.float32)

    xb = x_ref[...].astype(jnp.bfloat16)
    w13 = w13_ref[0]
    h1 = jnp.dot(xb, w13.T, preferred_element_type=jnp.float32)
    gate, up = jnp.split(h1, 2, axis=-1)
    h2 = (jax.nn.sigmoid(gate) * gate * up).astype(jnp.bfloat16)
    o = jnp.dot(h2, w2_ref[0].T, preferred_element_type=jnp.float32)
    coef = coef_ref[...]
    eidx = lax.broadcasted_iota(jnp.int32, coef.shape, 1)
    coef_e = jnp.sum(jnp.where(eidx == e, coef, 0.0), axis=-1, keepdims=True)
    out_ref[...] += coef_e * o


def kernel(hidden_states, gate_w, w13, w2, shared_w13, shared_w2):
    w13_b = w13.astype(jnp.bfloat16)
    w2_b = w2.astype(jnp.bfloat16)
    sw13_b = shared_w13.astype(jnp.bfloat16)
    sw2_b = shared_w2.astype(jnp.bfloat16)

    grid = (T // BT, E)
    out = pl.pallas_call(
        _moe_body,
        grid=grid,
        in_specs=[
            pl.BlockSpec((BT, H), lambda t, e: (t, 0)),
            pl.BlockSpec((E, H), lambda t, e: (0, 0)),
            pl.BlockSpec((1, 2 * F, H), lambda t, e: (e, 0, 0)),
            pl.BlockSpec((1, H, F), lambda t, e: (e, 0, 0)),
            pl.BlockSpec((2 * SF, H), lambda t, e: (0, 0)),
            pl.BlockSpec((H, SF), lambda t, e: (0, 0)),
        ],
        out_specs=pl.BlockSpec((BT, H), lambda t, e: (t, 0)),
        out_shape=jax.ShapeDtypeStruct((T, H), jnp.float32),
        scratch_shapes=[pltpu.VMEM((BT, E), jnp.float32)],
        compiler_params=pltpu.CompilerParams(
            dimension_semantics=("parallel", "arbitrary"),
        ),
    )(hidden_states, gate_w, w13_b, w2_b, sw13_b, sw2_b)
    return out
